# baseline (speedup 1.0000x reference)
import jax
import jax.numpy as jnp
from jax import lax
from jax.experimental import pallas as pl
from jax.experimental.pallas import tpu as pltpu
from jax.experimental.pallas import tpu_sc as plsc

VOCAB = 2495767
EMBED = 64
BATCH = 16384
NNEG = 20
NPAIR = NNEG + 1
SW = 32
NW = 32
BPW = BATCH // NW
TRING = 2
HALF = BPW // 2


def _sc_body(tc_idx_hbm, n_idx_hbm, temb_t, cflat, out_hbm,
             tc_idx_v, nv, tstage, trows, cstage, scores_v, s_tv, s_cv, sem):
    wid = lax.axis_index("s") * 2 + lax.axis_index("c")
    lane = lax.iota(jnp.int32, 16)

    pltpu.sync_copy(tc_idx_hbm.at[wid], tc_idx_v)
    pltpu.sync_copy(n_idx_hbm.at[wid], nv)

    def stage_idx(i, carry):
        ivt = tc_idx_v[i >> 3, pl.ds((i & 7) * 16, 16)]
        ivc = tc_idx_v[4 + (i >> 3), pl.ds((i & 7) * 16, 16)]
        for j in range(16):
            s_tv[i * 16 + j] = ivt[j]
            s_cv[i * 16 + j] = ivc[j]
        return carry

    lax.fori_loop(0, BPW // 16, stage_idx, 0)

    for h in range(2):
        base = h * HALF

        def phase_t(k, carry, base=base):
            @pl.when(k >= TRING)
            def _drain_extract():
                m = k - TRING
                q = m & (TRING - 1)
                pltpu.make_async_copy(temb_t.at[:, pl.ds(0, 128)],
                                      tstage.at[pl.ds(0, 64), :], sem).wait()
                c = s_tv[base + m] & 127
                cvec = jnp.full((16,), 0, jnp.int32) + c
                for u, off in enumerate((0, 1, 32, 33)):
                    rows = q * 64 + off + 2 * lane
                    tg = plsc.load_gather(tstage, [rows, cvec])
                    trows[pl.ds(m * EMBED + u * 16, 16)] = tg

            @pl.when(k < HALF)
            def _fire():
                q = k & (TRING - 1)
                v = s_tv[base + k]
                pltpu.async_copy(temb_t.at[:, pl.ds((v >> 7) * 128, 128)],
                                 tstage.at[pl.ds(q * 64, 64), :], sem)

            return carry

        lax.fori_loop(0, HALF + TRING, phase_t, 0)

        def phase_c(b, carry, base=base):
            @pl.when(b < HALF)
            def _fire():
                s = b & 1
                g = base + b
                vc = s_cv[g]
                pltpu.async_copy(cflat.at[pl.ds((vc >> 2) * 256, 256)],
                                 cstage.at[pl.ds(s * NPAIR * 256, 256)],
                                 sem)
                iv1 = nv[g >> 2, pl.ds((g & 3) * 32, 16)]
                iv2 = nv[g >> 2, pl.ds((g & 3) * 32 + 16, 16)]
                for n in range(NNEG):
                    vn = iv1[n] if n < 16 else iv2[n - 16]
                    pltpu.async_copy(
                        cflat.at[pl.ds((vn >> 2) * 256, 256)],
                        cstage.at[pl.ds((s * NPAIR + 1 + n) * 256, 256)],
                        sem)

            @pl.when(b > 0)
            def _drain_compute():
                m = b - 1
                s = m & 1
                pltpu.make_async_copy(
                    cflat.at[pl.ds(0, NPAIR * 256)],
                    cstage.at[pl.ds(0, NPAIR * 256)], sem).wait()
                t0 = trows[pl.ds(m * EMBED, 16)]
                t1 = trows[pl.ds(m * EMBED + 16, 16)]
                t2 = trows[pl.ds(m * EMBED + 32, 16)]
                t3 = trows[pl.ds(m * EMBED + 48, 16)]

                g = base + m
                iv1 = nv[g >> 2, pl.ds((g & 3) * 32, 16)]
                iv2 = nv[g >> 2, pl.ds((g & 3) * 32 + 16, 16)]

                def dot(p, v, s=s, t0=t0, t1=t1, t2=t2, t3=t3):
                    bb = (s * NPAIR + p) * 256 + (v & 3) * EMBED
                    a1, b1 = plsc.unpack(cstage[pl.ds(bb, 32)],
                                         format=plsc.PackFormat.INTERLEAVED)
                    a2, b2 = plsc.unpack(cstage[pl.ds(bb + 32, 32)],
                                         format=plsc.PackFormat.INTERLEAVED)
                    return jnp.sum(t0 * a1 + t1 * b1 + t2 * a2 + t3 * b2)

                lo = jnp.where(lane == 0, dot(0, s_cv[g]), 0.0)
                hi = jnp.zeros((16,), jnp.float32)
                for n in range(NNEG):
                    vn = iv1[n] if n < 16 else iv2[n - 16]
                    sneg = -dot(1 + n, vn)
                    if n < 15:
                        lo = jnp.where(lane == n + 1, sneg, lo)
                    else:
                        hi = jnp.where(lane == n - 15, sneg, hi)
                scores_v[pl.ds(m * SW, 16)] = lo
                scores_v[pl.ds(m * SW + 16, 16)] = hi

            return carry

        lax.fori_loop(0, HALF + 1, phase_c, 0)

        pltpu.sync_copy(
            scores_v,
            out_hbm.at[pl.ds((wid * BPW + base) * SW, HALF * SW)])


_sc_scores = pl.kernel(
    _sc_body,
    out_type=jax.ShapeDtypeStruct((BATCH * SW,), jnp.float32),
    mesh=plsc.VectorSubcoreMesh(core_axis_name="c", subcore_axis_name="s"),
    compiler_params=pltpu.CompilerParams(
        needs_layout_passes=False, use_tc_tiling_on_sc=True),
    scratch_types=[
        pltpu.VMEM((8, 128), jnp.int32),
        pltpu.VMEM((BPW // 4, 128), jnp.int32),
        pltpu.VMEM((TRING * 64, 128), jnp.float32),
        pltpu.VMEM((HALF * EMBED,), jnp.float32),
        pltpu.VMEM((2 * NPAIR * 256,), jnp.bfloat16),
        pltpu.VMEM((HALF * SW,), jnp.float32),
        pltpu.SMEM((BPW,), jnp.int32),
        pltpu.SMEM((BPW,), jnp.int32),
        pltpu.SemaphoreType.DMA,
    ],
)


def _loss_body(s_ref, o_ref):
    x = s_ref[...]
    col = lax.broadcasted_iota(jnp.int32, x.shape, 1)
    valid = (col % SW) < NPAIR
    y = jnp.where(valid, jax.nn.log_sigmoid(x), 0.0)
    o_ref[0, 0] = -jnp.sum(y) / BATCH


_tc_loss = pl.pallas_call(
    _loss_body,
    out_shape=jax.ShapeDtypeStruct((1, 1), jnp.float32),
    out_specs=pl.BlockSpec(memory_space=pltpu.SMEM),
)


@jax.jit
def kernel(target, context, negatives, target_emb, context_emb):
    tc_idx = jnp.concatenate(
        [target.astype(jnp.int32).reshape(NW, 4, 128),
         context.astype(jnp.int32).reshape(NW, 4, 128)], axis=1)
    n_idx = jnp.pad(negatives.astype(jnp.int32), ((0, 0), (0, SW - NNEG))
                    ).reshape(NW, BPW // 4, 128)
    cflat = context_emb.reshape(-1).astype(jnp.bfloat16)
    scores = _sc_scores(tc_idx, n_idx, target_emb.T, cflat)
    loss = _tc_loss(scores.reshape(BATCH * SW // 128, 128))
    return loss[0, 0]


# speedup vs baseline: 1.2586x; 1.2586x over previous
import jax
import jax.numpy as jnp
from jax import lax
from jax.experimental import pallas as pl
from jax.experimental.pallas import tpu as pltpu
from jax.experimental.pallas import tpu_sc as plsc

VOCAB = 2495767
EMBED = 64
BATCH = 16384
NNEG = 20
NPAIR = NNEG + 1
SW = 32
NW = 32
BPW = BATCH // NW
TRING = 2
HALF = BPW // 2

_params = pltpu.CompilerParams(
    needs_layout_passes=False, use_tc_tiling_on_sc=True)
_mesh = dict(core_axis_name="c", subcore_axis_name="s")


def _sc_a_body(t_idx_hbm, temb_t, trows_hbm,
               t_idx_v, tstage, trows_v, s_tv, sem):
    wid = lax.axis_index("s") * 2 + lax.axis_index("c")
    lane = lax.iota(jnp.int32, 16)

    pltpu.sync_copy(t_idx_hbm.at[wid], t_idx_v)

    def stage_idx(i, carry):
        iv = t_idx_v[i >> 3, pl.ds((i & 7) * 16, 16)]
        for j in range(16):
            s_tv[i * 16 + j] = iv[j]
        return carry

    lax.fori_loop(0, BPW // 16, stage_idx, 0)

    for h in range(2):
        base = h * HALF

        def phase_t(k, carry, base=base):
            @pl.when(k >= TRING)
            def _drain_extract():
                m = k - TRING
                q = m & (TRING - 1)
                pltpu.make_async_copy(temb_t.at[:, pl.ds(0, 128)],
                                      tstage.at[pl.ds(0, 64), :], sem).wait()
                c = s_tv[base + m] & 127
                cvec = jnp.full((16,), 0, jnp.int32) + c
                for u, off in enumerate((0, 1, 32, 33)):
                    rows = q * 64 + off + 2 * lane
                    tg = plsc.load_gather(tstage, [rows, cvec])
                    trows_v[pl.ds(m * EMBED + u * 16, 16)] = tg

            @pl.when(k < HALF)
            def _fire():
                q = k & (TRING - 1)
                v = s_tv[base + k]
                pltpu.async_copy(temb_t.at[:, pl.ds((v >> 7) * 128, 128)],
                                 tstage.at[pl.ds(q * 64, 64), :], sem)

            return carry

        lax.fori_loop(0, HALF + TRING, phase_t, 0)
        pltpu.sync_copy(
            trows_v,
            trows_hbm.at[pl.ds((wid * BPW + base) * EMBED, HALF * EMBED)])


_sc_trows = pl.kernel(
    _sc_a_body,
    out_type=jax.ShapeDtypeStruct((BATCH * EMBED,), jnp.float32),
    mesh=plsc.VectorSubcoreMesh(**_mesh),
    compiler_params=_params,
    scratch_types=[
        pltpu.VMEM((4, 128), jnp.int32),
        pltpu.VMEM((TRING * 64, 128), jnp.float32),
        pltpu.VMEM((HALF * EMBED,), jnp.float32),
        pltpu.SMEM((BPW,), jnp.int32),
        pltpu.SemaphoreType.DMA,
    ],
)


def _sc_b_body(c_idx_hbm, n_idx_hbm, cemb16, trows_hbm, out_hbm,
               c_idx_v, nv, trows_v, cstage, scores_v, s_cv, sem):
    wid = lax.axis_index("s") * 2 + lax.axis_index("c")
    lane = lax.iota(jnp.int32, 16)

    pltpu.sync_copy(c_idx_hbm.at[wid], c_idx_v)
    pltpu.sync_copy(n_idx_hbm.at[wid], nv)
    pltpu.sync_copy(trows_hbm.at[pl.ds(wid * BPW * EMBED, BPW * EMBED)],
                    trows_v)

    def stage_idx(i, carry):
        iv = c_idx_v[i >> 3, pl.ds((i & 7) * 16, 16)]
        for j in range(16):
            s_cv[i * 16 + j] = iv[j]
        return carry

    lax.fori_loop(0, BPW // 16, stage_idx, 0)

    def phase_c(b, carry):
        @pl.when(b < BPW)
        def _fire():
            s = b & 1
            vc = s_cv[b]
            pltpu.async_copy(cemb16.at[pl.ds((vc >> 4) * 16, 16), :],
                             cstage.at[pl.ds(s * NPAIR * 16, 16), :], sem)
            iv1 = nv[b >> 2, pl.ds((b & 3) * 32, 16)]
            iv2 = nv[b >> 2, pl.ds((b & 3) * 32 + 16, 16)]
            for n in range(NNEG):
                vn = iv1[n] if n < 16 else iv2[n - 16]
                pltpu.async_copy(
                    cemb16.at[pl.ds((vn >> 4) * 16, 16), :],
                    cstage.at[pl.ds((s * NPAIR + 1 + n) * 16, 16), :], sem)

        @pl.when(b > 0)
        def _drain_compute():
            m = b - 1
            s = m & 1
            pltpu.make_async_copy(cemb16.at[pl.ds(0, NPAIR * 16), :],
                                  cstage.at[pl.ds(0, NPAIR * 16), :],
                                  sem).wait()
            t0 = trows_v[pl.ds(m * EMBED, 16)]
            t1 = trows_v[pl.ds(m * EMBED + 16, 16)]
            t2 = trows_v[pl.ds(m * EMBED + 32, 16)]
            t3 = trows_v[pl.ds(m * EMBED + 48, 16)]
            iv1 = nv[m >> 2, pl.ds((m & 3) * 32, 16)]
            iv2 = nv[m >> 2, pl.ds((m & 3) * 32 + 16, 16)]

            def dot(p, v, s=s, t0=t0, t1=t1, t2=t2, t3=t3):
                row = (s * NPAIR + p) * 16 + (v & 15)
                a1, b1 = plsc.unpack(cstage[row, pl.ds(0, 32)],
                                     format=plsc.PackFormat.INTERLEAVED)
                a2, b2 = plsc.unpack(cstage[row, pl.ds(32, 32)],
                                     format=plsc.PackFormat.INTERLEAVED)
                return jnp.sum(t0 * a1 + t1 * b1 + t2 * a2 + t3 * b2)

            lo = jnp.where(lane == 0, dot(0, s_cv[m]), 0.0)
            hi = jnp.zeros((16,), jnp.float32)
            for n in range(NNEG):
                vn = iv1[n] if n < 16 else iv2[n - 16]
                sneg = -dot(1 + n, vn)
                if n < 15:
                    lo = jnp.where(lane == n + 1, sneg, lo)
                else:
                    hi = jnp.where(lane == n - 15, sneg, hi)
            scores_v[pl.ds(m * SW, 16)] = lo
            scores_v[pl.ds(m * SW + 16, 16)] = hi

        return carry

    lax.fori_loop(0, BPW + 1, phase_c, 0)

    pltpu.sync_copy(scores_v, out_hbm.at[pl.ds(wid * BPW * SW, BPW * SW)])


_sc_scores = pl.kernel(
    _sc_b_body,
    out_type=jax.ShapeDtypeStruct((BATCH * SW,), jnp.float32),
    mesh=plsc.VectorSubcoreMesh(**_mesh),
    compiler_params=_params,
    scratch_types=[
        pltpu.VMEM((4, 128), jnp.int32),
        pltpu.VMEM((BPW // 4, 128), jnp.int32),
        pltpu.VMEM((BPW * EMBED,), jnp.float32),
        pltpu.VMEM((2 * NPAIR * 16, EMBED), jnp.bfloat16),
        pltpu.VMEM((BPW * SW,), jnp.float32),
        pltpu.SMEM((BPW,), jnp.int32),
        pltpu.SemaphoreType.DMA,
    ],
)


def _loss_body(s_ref, o_ref):
    x = s_ref[...]
    col = lax.broadcasted_iota(jnp.int32, x.shape, 1)
    valid = (col % SW) < NPAIR
    y = jnp.where(valid, jax.nn.log_sigmoid(x), 0.0)
    o_ref[0, 0] = -jnp.sum(y) / BATCH


_tc_loss = pl.pallas_call(
    _loss_body,
    out_shape=jax.ShapeDtypeStruct((1, 1), jnp.float32),
    out_specs=pl.BlockSpec(memory_space=pltpu.SMEM),
)


@jax.jit
def kernel(target, context, negatives, target_emb, context_emb):
    t_idx = target.astype(jnp.int32).reshape(NW, 4, 128)
    c_idx = context.astype(jnp.int32).reshape(NW, 4, 128)
    n_idx = jnp.pad(negatives.astype(jnp.int32), ((0, 0), (0, SW - NNEG))
                    ).reshape(NW, BPW // 4, 128)
    cemb16 = context_emb.astype(jnp.bfloat16)
    trows = _sc_trows(t_idx, target_emb.T)
    scores = _sc_scores(c_idx, n_idx, cemb16, trows)
    loss = _tc_loss(scores.reshape(BATCH * SW // 128, 128))
    return loss[0, 0]


# speedup vs baseline: 1.3373x; 1.0625x over previous
import jax
import jax.numpy as jnp
from jax import lax
from jax.experimental import pallas as pl
from jax.experimental.pallas import tpu as pltpu
from jax.experimental.pallas import tpu_sc as plsc

VOCAB = 2495767
EMBED = 64
BATCH = 16384
NNEG = 20
NPAIR = NNEG + 1
SW = 32
NW = 32
BPW = BATCH // NW
TRING = 2
HALF = BPW // 2

_params = pltpu.CompilerParams(
    needs_layout_passes=False, use_tc_tiling_on_sc=True)
_mesh = dict(core_axis_name="c", subcore_axis_name="s")


def _sc_a_body(t_idx_hbm, temb_t, trows_hbm,
               t_idx_v, tstage, trows_v, s_tv, sem):
    wid = lax.axis_index("s") * 2 + lax.axis_index("c")
    lane = lax.iota(jnp.int32, 16)

    pltpu.sync_copy(t_idx_hbm.at[wid], t_idx_v)

    def stage_idx(i, carry):
        iv = t_idx_v[i >> 3, pl.ds((i & 7) * 16, 16)]
        for j in range(16):
            s_tv[i * 16 + j] = iv[j]
        return carry

    lax.fori_loop(0, BPW // 16, stage_idx, 0)

    for h in range(2):
        base = h * HALF

        def phase_t(k, carry, base=base):
            @pl.when(k >= TRING)
            def _drain_extract():
                m = k - TRING
                q = m & (TRING - 1)
                pltpu.make_async_copy(temb_t.at[:, pl.ds(0, 128)],
                                      tstage.at[pl.ds(0, 64), :], sem).wait()
                c = s_tv[base + m] & 127
                cvec = jnp.full((16,), 0, jnp.int32) + c
                for u, off in enumerate((0, 16, 32, 48)):
                    rows = q * 64 + off + lane
                    tg = plsc.load_gather(tstage, [rows, cvec])
                    trows_v[pl.ds(m * EMBED + u * 16, 16)] = tg

            @pl.when(k < HALF)
            def _fire():
                q = k & (TRING - 1)
                v = s_tv[base + k]
                pltpu.async_copy(temb_t.at[:, pl.ds((v >> 7) * 128, 128)],
                                 tstage.at[pl.ds(q * 64, 64), :], sem)

            return carry

        lax.fori_loop(0, HALF + TRING, phase_t, 0)
        pltpu.sync_copy(
            trows_v,
            trows_hbm.at[pl.ds((wid * BPW + base) * EMBED, HALF * EMBED)])


_sc_trows = pl.kernel(
    _sc_a_body,
    out_type=jax.ShapeDtypeStruct((BATCH * EMBED,), jnp.float32),
    mesh=plsc.VectorSubcoreMesh(**_mesh),
    compiler_params=_params,
    scratch_types=[
        pltpu.VMEM((4, 128), jnp.int32),
        pltpu.VMEM((TRING * 64, 128), jnp.float32),
        pltpu.VMEM((HALF * EMBED,), jnp.float32),
        pltpu.SMEM((BPW,), jnp.int32),
        pltpu.SemaphoreType.DMA,
    ],
)


def _sc_b_body(c_idx_hbm, n_idx_hbm, cemb, trows_hbm, out_hbm,
               c_idx_v, nv, trows_v, cstage, scores_v, s_cv, sem):
    wid = lax.axis_index("s") * 2 + lax.axis_index("c")
    lane = lax.iota(jnp.int32, 16)

    pltpu.sync_copy(c_idx_hbm.at[wid], c_idx_v)
    pltpu.sync_copy(n_idx_hbm.at[wid], nv)
    pltpu.sync_copy(trows_hbm.at[pl.ds(wid * BPW * EMBED, BPW * EMBED)],
                    trows_v)

    def stage_idx(i, carry):
        iv = c_idx_v[i >> 3, pl.ds((i & 7) * 16, 16)]
        for j in range(16):
            s_cv[i * 16 + j] = iv[j]
        return carry

    lax.fori_loop(0, BPW // 16, stage_idx, 0)

    def phase_c(b, carry):
        @pl.when(b < BPW)
        def _fire():
            s = b & 1
            vc = s_cv[b]
            pltpu.async_copy(cemb.at[pl.ds((vc >> 3) * 8, 8), :],
                             cstage.at[pl.ds(s * NPAIR * 8, 8), :], sem)
            iv1 = nv[b >> 2, pl.ds((b & 3) * 32, 16)]
            iv2 = nv[b >> 2, pl.ds((b & 3) * 32 + 16, 16)]
            for n in range(NNEG):
                vn = iv1[n] if n < 16 else iv2[n - 16]
                pltpu.async_copy(
                    cemb.at[pl.ds((vn >> 3) * 8, 8), :],
                    cstage.at[pl.ds((s * NPAIR + 1 + n) * 8, 8), :], sem)

        @pl.when(b > 0)
        def _drain_compute():
            m = b - 1
            s = m & 1
            pltpu.make_async_copy(cemb.at[pl.ds(0, NPAIR * 8), :],
                                  cstage.at[pl.ds(0, NPAIR * 8), :],
                                  sem).wait()
            t0 = trows_v[pl.ds(m * EMBED, 16)]
            t1 = trows_v[pl.ds(m * EMBED + 16, 16)]
            t2 = trows_v[pl.ds(m * EMBED + 32, 16)]
            t3 = trows_v[pl.ds(m * EMBED + 48, 16)]
            iv1 = nv[m >> 2, pl.ds((m & 3) * 32, 16)]
            iv2 = nv[m >> 2, pl.ds((m & 3) * 32 + 16, 16)]

            def dot(p, v, s=s, t0=t0, t1=t1, t2=t2, t3=t3):
                row = (s * NPAIR + p) * 8 + (v & 7)
                return jnp.sum(t0 * cstage[row, pl.ds(0, 16)]
                               + t1 * cstage[row, pl.ds(16, 16)]
                               + t2 * cstage[row, pl.ds(32, 16)]
                               + t3 * cstage[row, pl.ds(48, 16)])

            lo = jnp.where(lane == 0, dot(0, s_cv[m]), 0.0)
            hi = jnp.zeros((16,), jnp.float32)
            for n in range(NNEG):
                vn = iv1[n] if n < 16 else iv2[n - 16]
                sneg = -dot(1 + n, vn)
                if n < 15:
                    lo = jnp.where(lane == n + 1, sneg, lo)
                else:
                    hi = jnp.where(lane == n - 15, sneg, hi)
            scores_v[pl.ds(m * SW, 16)] = lo
            scores_v[pl.ds(m * SW + 16, 16)] = hi

        return carry

    lax.fori_loop(0, BPW + 1, phase_c, 0)

    pltpu.sync_copy(scores_v, out_hbm.at[pl.ds(wid * BPW * SW, BPW * SW)])


_sc_scores = pl.kernel(
    _sc_b_body,
    out_type=jax.ShapeDtypeStruct((BATCH * SW,), jnp.float32),
    mesh=plsc.VectorSubcoreMesh(**_mesh),
    compiler_params=_params,
    scratch_types=[
        pltpu.VMEM((4, 128), jnp.int32),
        pltpu.VMEM((BPW // 4, 128), jnp.int32),
        pltpu.VMEM((BPW * EMBED,), jnp.float32),
        pltpu.VMEM((2 * NPAIR * 8, EMBED), jnp.float32),
        pltpu.VMEM((BPW * SW,), jnp.float32),
        pltpu.SMEM((BPW,), jnp.int32),
        pltpu.SemaphoreType.DMA,
    ],
)


def _loss_body(s_ref, o_ref):
    x = s_ref[...]
    col = lax.broadcasted_iota(jnp.int32, x.shape, 1)
    valid = (col % SW) < NPAIR
    y = jnp.where(valid, jax.nn.log_sigmoid(x), 0.0)
    o_ref[0, 0] = -jnp.sum(y) / BATCH


_tc_loss = pl.pallas_call(
    _loss_body,
    out_shape=jax.ShapeDtypeStruct((1, 1), jnp.float32),
    out_specs=pl.BlockSpec(memory_space=pltpu.SMEM),
)


@jax.jit
def kernel(target, context, negatives, target_emb, context_emb):
    t_idx = target.astype(jnp.int32).reshape(NW, 4, 128)
    c_idx = context.astype(jnp.int32).reshape(NW, 4, 128)
    n_idx = jnp.pad(negatives.astype(jnp.int32), ((0, 0), (0, SW - NNEG))
                    ).reshape(NW, BPW // 4, 128)
    trows = _sc_trows(t_idx, target_emb.T)
    scores = _sc_scores(c_idx, n_idx, context_emb, trows)
    loss = _tc_loss(scores.reshape(BATCH * SW // 128, 128))
    return loss[0, 0]


# speedup vs baseline: 1.4453x; 1.0807x over previous
import jax
import jax.numpy as jnp
from jax import lax
from jax.experimental import pallas as pl
from jax.experimental.pallas import tpu as pltpu
from jax.experimental.pallas import tpu_sc as plsc

VOCAB = 2495767
EMBED = 64
BATCH = 16384
NNEG = 20
NPAIR = NNEG + 1
SW = 32
NW = 32
BPW = BATCH // NW
TRING = 2
HALF = BPW // 2

_params = pltpu.CompilerParams(
    needs_layout_passes=False, use_tc_tiling_on_sc=True)
_mesh = dict(core_axis_name="c", subcore_axis_name="s")


def _sc_a_body(t_idx_hbm, temb_t, trows_hbm,
               t_idx_v, tstage, trows_v, s_tv, sem):
    wid = lax.axis_index("s") * 2 + lax.axis_index("c")
    lane = lax.iota(jnp.int32, 16)

    pltpu.sync_copy(t_idx_hbm.at[wid], t_idx_v)

    def stage_idx(i, carry):
        iv = t_idx_v[i >> 3, pl.ds((i & 7) * 16, 16)]
        for j in range(16):
            s_tv[i * 16 + j] = iv[j]
        return carry

    lax.fori_loop(0, BPW // 16, stage_idx, 0)

    for h in range(2):
        base = h * HALF

        def phase_t(k, carry, base=base):
            @pl.when(k >= TRING)
            def _drain_extract():
                m = k - TRING
                q = m & (TRING - 1)
                pltpu.make_async_copy(temb_t.at[:, pl.ds(0, 128)],
                                      tstage.at[pl.ds(0, 64), :], sem).wait()
                c = s_tv[base + m] & 127
                cvec = jnp.full((16,), 0, jnp.int32) + c
                for u, off in enumerate((0, 16, 32, 48)):
                    rows = q * 64 + off + lane
                    tg = plsc.load_gather(tstage, [rows, cvec])
                    trows_v[pl.ds(m * EMBED + u * 16, 16)] = tg

            @pl.when(k < HALF)
            def _fire():
                q = k & (TRING - 1)
                v = s_tv[base + k]
                pltpu.async_copy(temb_t.at[:, pl.ds((v >> 7) * 128, 128)],
                                 tstage.at[pl.ds(q * 64, 64), :], sem)

            return carry

        lax.fori_loop(0, HALF + TRING, phase_t, 0)
        pltpu.sync_copy(
            trows_v,
            trows_hbm.at[pl.ds((wid * BPW + base) * EMBED, HALF * EMBED)])


_sc_trows = pl.kernel(
    _sc_a_body,
    out_type=jax.ShapeDtypeStruct((BATCH * EMBED,), jnp.float32),
    mesh=plsc.VectorSubcoreMesh(**_mesh),
    compiler_params=_params,
    scratch_types=[
        pltpu.VMEM((4, 128), jnp.int32),
        pltpu.VMEM((TRING * 64, 128), jnp.float32),
        pltpu.VMEM((HALF * EMBED,), jnp.float32),
        pltpu.SMEM((BPW,), jnp.int32),
        pltpu.SemaphoreType.DMA,
    ],
)


def _sc_b_body(c_idx_hbm, n_idx_hbm, cemb, trows_hbm, out_hbm,
               c_idx_v, nv, trows_v, cstage, scores_v, s_cv, sem):
    wid = lax.axis_index("s") * 2 + lax.axis_index("c")
    lane = lax.iota(jnp.int32, 16)

    pltpu.sync_copy(c_idx_hbm.at[wid], c_idx_v)
    pltpu.sync_copy(n_idx_hbm.at[wid], nv)
    pltpu.sync_copy(trows_hbm.at[pl.ds(wid * BPW * EMBED, BPW * EMBED)],
                    trows_v)

    def stage_idx(i, carry):
        iv = c_idx_v[i >> 3, pl.ds((i & 7) * 16, 16)]
        for j in range(16):
            s_cv[i * 16 + j] = iv[j]
        return carry

    lax.fori_loop(0, BPW // 16, stage_idx, 0)

    def phase_c(b, carry):
        @pl.when(b < BPW)
        def _fire():
            s = b % 3
            vc = s_cv[b]
            pltpu.async_copy(cemb.at[pl.ds((vc >> 3) * 8, 8), :],
                             cstage.at[pl.ds(s * NPAIR * 8, 8), :], sem)
            iv1 = nv[b >> 2, pl.ds((b & 3) * 32, 16)]
            iv2 = nv[b >> 2, pl.ds((b & 3) * 32 + 16, 16)]
            for n in range(NNEG):
                vn = iv1[n] if n < 16 else iv2[n - 16]
                pltpu.async_copy(
                    cemb.at[pl.ds((vn >> 3) * 8, 8), :],
                    cstage.at[pl.ds((s * NPAIR + 1 + n) * 8, 8), :], sem)

        @pl.when(b >= 2)
        def _drain_compute():
            m = b - 2
            s = m % 3
            pltpu.make_async_copy(cemb.at[pl.ds(0, NPAIR * 8), :],
                                  cstage.at[pl.ds(0, NPAIR * 8), :],
                                  sem).wait()
            t0 = trows_v[pl.ds(m * EMBED, 16)]
            t1 = trows_v[pl.ds(m * EMBED + 16, 16)]
            t2 = trows_v[pl.ds(m * EMBED + 32, 16)]
            t3 = trows_v[pl.ds(m * EMBED + 48, 16)]
            iv1 = nv[m >> 2, pl.ds((m & 3) * 32, 16)]
            iv2 = nv[m >> 2, pl.ds((m & 3) * 32 + 16, 16)]

            def dot(p, v, s=s, t0=t0, t1=t1, t2=t2, t3=t3):
                row = (s * NPAIR + p) * 8 + (v & 7)
                return jnp.sum(t0 * cstage[row, pl.ds(0, 16)]
                               + t1 * cstage[row, pl.ds(16, 16)]
                               + t2 * cstage[row, pl.ds(32, 16)]
                               + t3 * cstage[row, pl.ds(48, 16)])

            lo = jnp.where(lane == 0, dot(0, s_cv[m]), 0.0)
            hi = jnp.zeros((16,), jnp.float32)
            for n in range(NNEG):
                vn = iv1[n] if n < 16 else iv2[n - 16]
                sneg = -dot(1 + n, vn)
                if n < 15:
                    lo = jnp.where(lane == n + 1, sneg, lo)
                else:
                    hi = jnp.where(lane == n - 15, sneg, hi)
            scores_v[pl.ds(m * SW, 16)] = lo
            scores_v[pl.ds(m * SW + 16, 16)] = hi

        return carry

    lax.fori_loop(0, BPW + 2, phase_c, 0)

    pltpu.sync_copy(scores_v, out_hbm.at[pl.ds(wid * BPW * SW, BPW * SW)])


_sc_scores = pl.kernel(
    _sc_b_body,
    out_type=jax.ShapeDtypeStruct((BATCH * SW,), jnp.float32),
    mesh=plsc.VectorSubcoreMesh(**_mesh),
    compiler_params=_params,
    scratch_types=[
        pltpu.VMEM((4, 128), jnp.int32),
        pltpu.VMEM((BPW // 4, 128), jnp.int32),
        pltpu.VMEM((BPW * EMBED,), jnp.float32),
        pltpu.VMEM((3 * NPAIR * 8, EMBED), jnp.float32),
        pltpu.VMEM((BPW * SW,), jnp.float32),
        pltpu.SMEM((BPW,), jnp.int32),
        pltpu.SemaphoreType.DMA,
    ],
)


def _loss_body(s_ref, o_ref):
    x = s_ref[...]
    col = lax.broadcasted_iota(jnp.int32, x.shape, 1)
    valid = (col % SW) < NPAIR
    y = jnp.where(valid, jax.nn.log_sigmoid(x), 0.0)
    o_ref[0, 0] = -jnp.sum(y) / BATCH


_tc_loss = pl.pallas_call(
    _loss_body,
    out_shape=jax.ShapeDtypeStruct((1, 1), jnp.float32),
    out_specs=pl.BlockSpec(memory_space=pltpu.SMEM),
)


@jax.jit
def kernel(target, context, negatives, target_emb, context_emb):
    t_idx = target.astype(jnp.int32).reshape(NW, 4, 128)
    c_idx = context.astype(jnp.int32).reshape(NW, 4, 128)
    n_idx = jnp.pad(negatives.astype(jnp.int32), ((0, 0), (0, SW - NNEG))
                    ).reshape(NW, BPW // 4, 128)
    trows = _sc_trows(t_idx, target_emb.T)
    scores = _sc_scores(c_idx, n_idx, context_emb, trows)
    loss = _tc_loss(scores.reshape(BATCH * SW // 128, 128))
    return loss[0, 0]
